# baseline (speedup 1.0000x reference)
import jax
import jax.numpy as jnp
from jax import lax
from jax.experimental import pallas as pl
from jax.experimental.pallas import tpu as pltpu
from jax.experimental.pallas import tpu_sc as plsc

_VOCAB = 1000000
_D = 64
_B = 4096
_L = 200

_NC = 2
_NS = 16
_NW = _NC * _NS
_SEQ_W = _B // _NW
_SPC = 2
_NCH = _SEQ_W // _SPC
_LANES = 16
_SPLITS = ((0, 128), (128, 72))


def _sc_body(x, tbl, posf, out, idx_all, rows_v, pos_v,
             gsem0, gsem1, ssem0, ssem1):
  cid = lax.axis_index("c")
  sid = lax.axis_index("s")
  wid = sid * _NC + cid
  seq_base = wid * _SEQ_W

  gsems = (gsem0, gsem1)
  ssems = (ssem0, ssem1)

  pltpu.sync_copy(posf, pos_v)
  pltpu.sync_copy(x.at[pl.ds(seq_base, _SEQ_W)], idx_all)

  def issue_gather(c, b):
    for j in range(_SPC):
      for (o, n) in _SPLITS:
        pltpu.async_copy(
            tbl.at[idx_all.at[c * _SPC + j, pl.ds(o, n)]],
            rows_v.at[b, j, pl.ds(o, n)], gsems[b])

  def wait_gather(b):
    pltpu.make_async_copy(
        out.at[pl.ds(0, _SPC)], rows_v.at[b], gsems[b]).wait()

  def issue_store(c, b):
    pltpu.async_copy(rows_v.at[b],
                     out.at[pl.ds(seq_base + c * _SPC, _SPC)], ssems[b])

  def wait_store(b):
    pltpu.make_async_copy(rows_v.at[b], out.at[pl.ds(0, _SPC)],
                          ssems[b]).wait()

  def add_pos(b):
    @pl.loop(0, _L, unroll=2)
    def _row(r):
      for k in range(_D // _LANES):
        v = pos_v[r, pl.ds(k * _LANES, _LANES)]
        for j in range(_SPC):
          plsc.addupdate(rows_v.at[b, j, r, pl.ds(k * _LANES, _LANES)], v)

  issue_gather(0, 0)

  @pl.loop(0, _NCH // 2)
  def _pair(c2):
    c0 = c2 * 2
    for half in range(2):
      cc = c0 + half
      b = half
      nb = 1 - half

      @pl.when(cc > 0)
      def _():
        wait_store(nb)

      @pl.when(cc + 1 < _NCH)
      def _():
        issue_gather(cc + 1, nb)

      wait_gather(b)
      add_pos(b)
      issue_store(cc, b)

  wait_store(1)


@jax.jit
def _run(x, token_table, pos_table):
  mesh = plsc.VectorSubcoreMesh(
      core_axis_name="c", subcore_axis_name="s",
      num_cores=_NC, num_subcores=_NS)
  kern = pl.kernel(
      _sc_body,
      out_type=jax.ShapeDtypeStruct((_B, _L, _D), jnp.float32),
      mesh=mesh,
      scratch_types=[
          pltpu.VMEM((_SEQ_W, _L), jnp.int32),
          pltpu.VMEM((2, _SPC, _L, _D), jnp.float32),
          pltpu.VMEM((_L, _D), jnp.float32),
          pltpu.SemaphoreType.DMA,
          pltpu.SemaphoreType.DMA,
          pltpu.SemaphoreType.DMA,
          pltpu.SemaphoreType.DMA,
      ],
      compiler_params=pltpu.CompilerParams(use_tc_tiling_on_sc=False),
  )
  return kern(x, token_table, pos_table)


def kernel(x, token_table, pos_table):
  return _run(x, token_table, pos_table[:_L])
